# Initial kernel scaffold; baseline (speedup 1.0000x reference)
import jax
import jax.numpy as jnp
from jax.experimental import pallas as pl


def kernel(node_attr, edge_index, num_atoms, W_enc, b_enc, W_g0, b_g0, W_g1, b_g1, W_g2, b_g2, W_r1, b_r1, W_r2, b_r2, W_s1, b_s1, W_s2, b_s2, W_f, b_f):
    raise NotImplementedError("write your pallas kernel here")


# speedup vs baseline: 6.8377x; 6.8377x over previous
import functools

import jax
import jax.numpy as jnp
from jax import lax
from jax.experimental import pallas as pl
from jax.experimental.pallas import tpu as pltpu
from jax.experimental.pallas import tpu_sc as plsc

N = 10000
D = 128
E = 320000
CHUNK = 80
ROWS_PER_TILE = 125
EROWS = E // CHUNK
NZ = N // 16
DEG_W = 16

_mesh = plsc.VectorSubcoreMesh(core_axis_name="c", subcore_axis_name="s")
_sc_params = pltpu.CompilerParams(use_tc_tiling_on_sc=False)


@functools.partial(
    pl.kernel,
    out_type=jax.ShapeDtypeStruct((2, N, D), jnp.float32),
    mesh=_mesh,
    scratch_types=[
        pltpu.VMEM((ROWS_PER_TILE, CHUNK), jnp.int32),
        pltpu.VMEM((ROWS_PER_TILE, CHUNK), jnp.int32),
        pltpu.VMEM((CHUNK, D), jnp.float32),
        pltpu.VMEM_SHARED((N, D), jnp.float32),
    ],
    compiler_params=_sc_params,
)
def _sc_agg(h_hbm, dst_hbm, src_hbm, zeros_hbm, out_hbm,
            dst_v, src_v, rows_v, acc_sh):
    cid = lax.axis_index("c")
    sid = lax.axis_index("s")
    wid = sid * 2 + cid
    zbase = sid * NZ
    pltpu.sync_copy(zeros_hbm, acc_sh.at[pl.ds(zbase, NZ)])
    ebase = wid * ROWS_PER_TILE
    pltpu.sync_copy(dst_hbm.at[pl.ds(ebase, ROWS_PER_TILE)], dst_v)
    pltpu.sync_copy(src_hbm.at[pl.ds(ebase, ROWS_PER_TILE)], src_v)
    plsc.subcore_barrier()

    @pl.loop(0, ROWS_PER_TILE)
    def _(j):
        pltpu.sync_copy(h_hbm.at[dst_v.at[j]], rows_v)
        pltpu.sync_copy(rows_v, acc_sh.at[src_v.at[j]], add=True)

    plsc.subcore_barrier()
    pltpu.sync_copy(acc_sh.at[pl.ds(zbase, NZ)],
                    out_hbm.at[cid, pl.ds(zbase, NZ)])


@functools.partial(
    pl.kernel,
    out_type=jax.ShapeDtypeStruct((2, N, DEG_W), jnp.float32),
    mesh=_mesh,
    scratch_types=[
        pltpu.VMEM((ROWS_PER_TILE, CHUNK), jnp.int32),
        pltpu.VMEM((CHUNK, DEG_W), jnp.float32),
        pltpu.VMEM_SHARED((N, DEG_W), jnp.float32),
    ],
    compiler_params=_sc_params,
)
def _sc_deg(src_hbm, ones_hbm, zeros_hbm, out_hbm, src_v, ones_v, acc_sh):
    cid = lax.axis_index("c")
    sid = lax.axis_index("s")
    wid = sid * 2 + cid
    zbase = sid * NZ
    pltpu.sync_copy(zeros_hbm, acc_sh.at[pl.ds(zbase, NZ)])
    pltpu.sync_copy(ones_hbm, ones_v)
    ebase = wid * ROWS_PER_TILE
    pltpu.sync_copy(src_hbm.at[pl.ds(ebase, ROWS_PER_TILE)], src_v)
    plsc.subcore_barrier()

    @pl.loop(0, ROWS_PER_TILE)
    def _(j):
        pltpu.sync_copy(ones_v, acc_sh.at[src_v.at[j]], add=True)

    plsc.subcore_barrier()
    pltpu.sync_copy(acc_sh.at[pl.ds(zbase, NZ)],
                    out_hbm.at[cid, pl.ds(zbase, NZ)])


BR = 1000
GRID = (N // BR,)

def _leaky(x):
    return jnp.where(x >= 0, x, 0.01 * x)


def _mm(x, w, b):
    return jnp.dot(x, w, preferred_element_type=jnp.float32) + b


def _pre_body(x_ref, we, be, wg, bg, o_ref):
    x0 = _mm(x_ref[...], we[...], be[...])
    o_ref[...] = _mm(x0, wg[...], bg[...])


def _combine(h_ref, agg_ref, deg_ref):
    aggs = agg_ref[0] + agg_ref[1]
    deg = deg_ref[0, :, 0:1] + deg_ref[1, :, 0:1]
    return (h_ref[...] + aggs) * (1.0 / (deg + 1.0))


def _mid_body(h_ref, agg_ref, deg_ref, wg, bg, o_ref):
    o_ref[...] = _mm(_combine(h_ref, agg_ref, deg_ref), wg[...], bg[...])


def _post_body(h_ref, agg_ref, deg_ref, wr1, br1, wr2, br2,
               ws1, bs1, ws2, bs2, wf, bf, o_ref):
    x = _combine(h_ref, agg_ref, deg_ref)
    r = _leaky(_mm(_leaky(_mm(x, wr1[...], br1[...])), wr2[...], br2[...]))
    s = jax.nn.sigmoid(_mm(_leaky(_mm(x, ws1[...], bs1[...])), ws2[...], bs2[...]))
    o_ref[...] = _mm(r * s, wf[...], bf[...])


def _w_spec():
    return pl.BlockSpec((D, D), lambda i: (0, 0))


def _b_spec():
    return pl.BlockSpec((D,), lambda i: (0,))


_X_SPEC = pl.BlockSpec((BR, D), lambda i: (i, 0))
_AGG_SPEC = pl.BlockSpec((2, BR, D), lambda i: (0, i, 0))
_DEG_SPEC = pl.BlockSpec((2, BR, DEG_W), lambda i: (0, i, 0))


def _tc_pre(node_attr, W_enc, b_enc, W_g0, b_g0):
    return pl.pallas_call(
        _pre_body,
        grid=GRID,
        in_specs=[_X_SPEC, _w_spec(), _b_spec(), _w_spec(), _b_spec()],
        out_specs=_X_SPEC,
        out_shape=jax.ShapeDtypeStruct((N, D), jnp.float32),
    )(node_attr, W_enc, b_enc, W_g0, b_g0)


def _tc_mid(h, agg, deg, Wg, bg):
    return pl.pallas_call(
        _mid_body,
        grid=GRID,
        in_specs=[_X_SPEC, _AGG_SPEC, _DEG_SPEC, _w_spec(), _b_spec()],
        out_specs=_X_SPEC,
        out_shape=jax.ShapeDtypeStruct((N, D), jnp.float32),
    )(h, agg, deg, Wg, bg)


def _tc_post(h, agg, deg, W_r1, b_r1, W_r2, b_r2, W_s1, b_s1, W_s2, b_s2,
             W_f_pad, b_f_pad):
    return pl.pallas_call(
        _post_body,
        grid=GRID,
        in_specs=[_X_SPEC, _AGG_SPEC, _DEG_SPEC,
                  _w_spec(), _b_spec(), _w_spec(), _b_spec(),
                  _w_spec(), _b_spec(), _w_spec(), _b_spec(),
                  _w_spec(), _b_spec()],
        out_specs=_X_SPEC,
        out_shape=jax.ShapeDtypeStruct((N, D), jnp.float32),
    )(h, agg, deg, W_r1, b_r1, W_r2, b_r2, W_s1, b_s1, W_s2, b_s2,
      W_f_pad, b_f_pad)


def kernel(node_attr, edge_index, num_atoms,
           W_enc, b_enc, W_g0, b_g0, W_g1, b_g1, W_g2, b_g2,
           W_r1, b_r1, W_r2, b_r2, W_s1, b_s1, W_s2, b_s2, W_f, b_f):
    del num_atoms
    src2d = edge_index[0].reshape(EROWS, CHUNK)
    dst2d = edge_index[1].reshape(EROWS, CHUNK)
    zrows = jnp.zeros((NZ, D), jnp.float32)
    zrows_deg = jnp.zeros((NZ, DEG_W), jnp.float32)
    ones_deg = jnp.ones((CHUNK, DEG_W), jnp.float32)
    nclass = W_f.shape[1]
    W_f_pad = jnp.zeros((D, D), jnp.float32).at[:, :nclass].set(W_f)
    b_f_pad = jnp.zeros((D,), jnp.float32).at[:nclass].set(b_f)

    deg = _sc_deg(src2d, ones_deg, zrows_deg)
    h = _tc_pre(node_attr, W_enc, b_enc, W_g0, b_g0)
    agg = _sc_agg(h, dst2d, src2d, zrows)
    h = _tc_mid(h, agg, deg, W_g1, b_g1)
    agg = _sc_agg(h, dst2d, src2d, zrows)
    h = _tc_mid(h, agg, deg, W_g2, b_g2)
    agg = _sc_agg(h, dst2d, src2d, zrows)
    out = _tc_post(h, agg, deg, W_r1, b_r1, W_r2, b_r2,
                   W_s1, b_s1, W_s2, b_s2, W_f_pad, b_f_pad)
    return out[:, :nclass]
